# Initial kernel scaffold; baseline (speedup 1.0000x reference)
import jax
import jax.numpy as jnp
from jax.experimental import pallas as pl


def kernel(memory, source_nodes, tppr_scores, timestamps, node_times, edge_feat_table, k, W_fc1, b_fc1, W_fc2, b_fc2, W_fc1_src, b_fc1_src, W_fc2_src, b_fc2_src, time_w, time_b):
    raise NotImplementedError("write your pallas kernel here")


# speedup vs baseline: 1.0135x; 1.0135x over previous
import functools

import jax
import jax.numpy as jnp
from jax.experimental import pallas as pl

N = 50000
B = 1024
D = 172
TD = 100
ED = 172
F1 = 512
F2 = 128
K = 20
GB = 64


def _mlp_body(nbr_ref, te_ref, ef_ref, wf_ref, src_ref,
              W1a_ref, W1b_ref, W1c_ref, b1_ref, W2_ref, b2_ref,
              Ws1_ref, bs1_ref, Ws2_ref, bs2_ref, out_ref):
    acc = jnp.dot(nbr_ref[...], W1a_ref[...], preferred_element_type=jnp.float32)
    acc += jnp.dot(te_ref[...], W1b_ref[...], preferred_element_type=jnp.float32)
    acc += jnp.dot(ef_ref[...], W1c_ref[...], preferred_element_type=jnp.float32)
    a = jnp.maximum(acc + b1_ref[...], 0.0)
    z = jnp.dot(a, W2_ref[...], preferred_element_type=jnp.float32) + b2_ref[...]
    zw = z * wf_ref[...]
    rows = jax.lax.broadcasted_iota(jnp.int32, (GB, GB * K), 1) // K
    gids = jax.lax.broadcasted_iota(jnp.int32, (GB, GB * K), 0)
    sel = (rows == gids).astype(jnp.float32)
    nbr_emb = jnp.dot(sel, zw, preferred_element_type=jnp.float32)
    s = src_ref[...]
    sa = jnp.maximum(jnp.dot(s, Ws1_ref[...], preferred_element_type=jnp.float32)
                     + bs1_ref[...], 0.0)
    out_ref[...] = (nbr_emb
                    + jnp.dot(sa, Ws2_ref[...], preferred_element_type=jnp.float32)
                    + bs2_ref[...])


def _full(shape):
    return pl.BlockSpec(shape, lambda i: (0, 0))


@functools.partial(jax.jit, static_argnames=())
def _combine(nbr_feat, t_enc, e_feat, wf, src,
             W1a, W1b, W1c, b1, W2, b2, Ws1, bs1, Ws2, bs2):
    grid = (B // GB,)
    return pl.pallas_call(
        _mlp_body,
        grid=grid,
        in_specs=[
            pl.BlockSpec((GB * K, D), lambda i: (i, 0)),
            pl.BlockSpec((GB * K, TD), lambda i: (i, 0)),
            pl.BlockSpec((GB * K, ED), lambda i: (i, 0)),
            pl.BlockSpec((GB * K, 1), lambda i: (i, 0)),
            pl.BlockSpec((GB, D), lambda i: (i, 0)),
            _full((D, F1)), _full((TD, F1)), _full((ED, F1)), _full((1, F1)),
            _full((F1, F2)), _full((1, F2)),
            _full((D, F1)), _full((1, F1)), _full((F1, F2)), _full((1, F2)),
        ],
        out_specs=pl.BlockSpec((GB, F2), lambda i: (i, 0)),
        out_shape=jax.ShapeDtypeStruct((B, F2), jnp.float32),
    )(nbr_feat, t_enc, e_feat, wf, src,
      W1a, W1b, W1c, b1, W2, b2, Ws1, bs1, Ws2, bs2)


def kernel(memory, source_nodes, tppr_scores, timestamps, node_times,
           edge_feat_table, k, W_fc1, b_fc1, W_fc2, b_fc2,
           W_fc1_src, b_fc1_src, W_fc2_src, b_fc2_src, time_w, time_b):
    top_w, top_idx = jax.lax.top_k(tppr_scores, K)
    w = jax.nn.softmax(top_w, axis=-1)
    flat_idx = top_idx.reshape(-1)
    nbr_feat = jnp.take(memory, flat_idx, axis=0)
    e_feat = jnp.take(edge_feat_table, flat_idx, axis=0)
    dt = timestamps[:, None] - jnp.take(node_times, flat_idx).reshape(B, K)
    t_enc = jnp.cos(dt.reshape(-1)[:, None] * time_w + time_b)
    src = jnp.take(memory, source_nodes, axis=0)
    out = _combine(
        nbr_feat, t_enc, e_feat, w.reshape(-1, 1), src,
        W_fc1[:D], W_fc1[D:D + TD], W_fc1[D + TD:], b_fc1.reshape(1, -1),
        W_fc2, b_fc2.reshape(1, -1),
        W_fc1_src, b_fc1_src.reshape(1, -1), W_fc2_src, b_fc2_src.reshape(1, -1))
    return out
